# Initial kernel scaffold; baseline (speedup 1.0000x reference)
import jax
import jax.numpy as jnp
from jax.experimental import pallas as pl


def kernel(x, edge_index, pseudo, W1, root1, b1, W2, root2, b2, lw1, lb1, lw2, lb2):
    raise NotImplementedError("write your pallas kernel here")


# speedup vs baseline: 1.0017x; 1.0017x over previous
import jax
import jax.numpy as jnp
from jax.experimental import pallas as pl

KS = 3
DIM = 3
NK = KS ** DIM


def _spline_conv(x, edge_index, pseudo, W, root, bias):
    src = edge_index[0]
    dst = edge_index[1]
    n = x.shape[0]
    u = pseudo * (KS - 1)
    i0 = jnp.clip(jnp.floor(u).astype(jnp.int32), 0, KS - 2)
    frac = u - i0.astype(jnp.float32)
    xW = jnp.einsum('ni,kio->nko', x, W)
    strides = jnp.array([KS ** d for d in range(DIM)], dtype=jnp.int32)
    msg = jnp.zeros((src.shape[0], W.shape[2]), dtype=jnp.float32)
    for c in range(2 ** DIM):
        off = jnp.array([(c >> d) & 1 for d in range(DIM)], dtype=jnp.int32)
        idx = i0 + off[None, :]
        b = jnp.prod(jnp.where(off[None, :] == 1, frac, 1.0 - frac), axis=1)
        kidx = jnp.sum(idx * strides[None, :], axis=1)
        msg = msg + b[:, None] * xW[src, kidx]
    agg = jax.ops.segment_sum(msg, dst, num_segments=n)
    deg = jax.ops.segment_sum(jnp.ones((dst.shape[0],), jnp.float32), dst, num_segments=n)
    agg = agg / jnp.clip(deg, 1.0)[:, None]
    return agg + x @ root + bias


def _elu(v):
    return jnp.where(v > 0, v, jnp.exp(jnp.minimum(v, 0.0)) - 1.0)


def _mlp_body(h_ref, lw1_ref, lb1_ref, lw2_ref, lb2_ref, o_ref):
    h = h_ref[...]
    h = _elu(h @ lw1_ref[...] + lb1_ref[...])
    h = h @ lw2_ref[...] + lb2_ref[...]
    m = jnp.max(h, axis=-1, keepdims=True)
    e = jnp.exp(h - m)
    o_ref[...] = (h - m) - jnp.log(jnp.sum(e, axis=-1, keepdims=True))


def kernel(x, edge_index, pseudo, W1, root1, b1, W2, root2, b2, lw1, lb1, lw2, lb2):
    h = jax.nn.elu(_spline_conv(x, edge_index, pseudo, W1, root1, b1))
    h = jax.nn.elu(_spline_conv(h, edge_index, pseudo, W2, root2, b2))
    n = h.shape[0]
    blk = 2000
    out = pl.pallas_call(
        _mlp_body,
        out_shape=jax.ShapeDtypeStruct((n, 4), jnp.float32),
        grid=(n // blk,),
        in_specs=[
            pl.BlockSpec((blk, 16), lambda i: (i, 0)),
            pl.BlockSpec((16, 32), lambda i: (0, 0)),
            pl.BlockSpec((32,), lambda i: (0,)),
            pl.BlockSpec((32, 4), lambda i: (0, 0)),
            pl.BlockSpec((4,), lambda i: (0,)),
        ],
        out_specs=pl.BlockSpec((blk, 4), lambda i: (i, 0)),
    )(h, lw1, lb1, lw2, lb2)
    return out


# speedup vs baseline: 2.7686x; 2.7638x over previous
import functools

import jax
import jax.numpy as jnp
from jax import lax
from jax.experimental import pallas as pl
from jax.experimental.pallas import tpu as pltpu
from jax.experimental.pallas import tpu_sc as plsc

N = 10000
E = 320000
KS = 3
DIM = 3
NK = KS ** DIM
LANE = 128
NR = E // LANE
NR_PAD = 2560
E_PAD = NR_PAD * LANE
N_PAD = 10240
NC = 2
NS = 16
ROWS_PER_TILE = NR_PAD // (NC * NS)
RCHUNK = 4
NPT = N_PAD // NS


def _elu(v):
    return jnp.where(v > 0, v, jnp.exp(jnp.minimum(v, 0.0)) - 1.0)


def _edge_prep_body(src_ref, p0_ref, p1_ref, p2_ref, fidx_ref, w_ref):
    src = src_ref[...]
    ps = (p0_ref[...], p1_ref[...], p2_ref[...])
    i0 = []
    fr = []
    for d in range(DIM):
        u = ps[d] * float(KS - 1)
        i0d = jnp.clip(jnp.floor(u).astype(jnp.int32), 0, KS - 2)
        i0.append(i0d)
        fr.append(u - i0d.astype(jnp.float32))
    base = src * NK + i0[0] + i0[1] * 3 + i0[2] * 9
    for c in range(8):
        offs = [(c >> d) & 1 for d in range(DIM)]
        kadd = offs[0] + offs[1] * 3 + offs[2] * 9
        w = jnp.ones_like(fr[0])
        for d in range(DIM):
            w = w * (fr[d] if offs[d] else (1.0 - fr[d]))
        fidx_ref[c] = base + kadd
        w_ref[c] = w


def _edge_prep(src2d, p02d, p12d, p22d):
    blk = 256
    grid = (NR_PAD // blk,)
    in_spec = pl.BlockSpec((blk, LANE), lambda i: (i, 0))
    out_spec = pl.BlockSpec((8, blk, LANE), lambda i: (0, i, 0))
    return pl.pallas_call(
        _edge_prep_body,
        grid=grid,
        in_specs=[in_spec] * 4,
        out_specs=[out_spec, out_spec],
        out_shape=[
            jax.ShapeDtypeStruct((8, NR_PAD, LANE), jnp.int32),
            jax.ShapeDtypeStruct((8, NR_PAD, LANE), jnp.float32),
        ],
    )(src2d, p02d, p12d, p22d)


def _mm1_body(x_ref, w_ref, o_ref):
    o_ref[...] = jnp.dot(x_ref[...], w_ref[...],
                         preferred_element_type=jnp.float32)


def _table1(x, w1r):
    blk = 1000
    return pl.pallas_call(
        _mm1_body,
        grid=(N // blk,),
        in_specs=[
            pl.BlockSpec((blk, 34), lambda i: (i, 0)),
            pl.BlockSpec((34, NK * 32), lambda i: (0, 0)),
        ],
        out_specs=pl.BlockSpec((blk, NK * 32), lambda i: (i, 0)),
        out_shape=jax.ShapeDtypeStruct((N, NK * 32), jnp.float32),
    )(x, w1r)


def _mid_body(a0_ref, a1_ref, d0_ref, d1_ref, x_ref, root_ref, b_ref,
              w2_ref, h1_ref, t2_ref):
    deg = d0_ref[...][:, :1] + d1_ref[...][:, :1]
    inv = 1.0 / jnp.maximum(deg, 1.0)
    agg = (a0_ref[...] + a1_ref[...]) * inv
    h1 = _elu(agg + jnp.dot(x_ref[...], root_ref[...],
                            preferred_element_type=jnp.float32) + b_ref[...])
    h1_ref[...] = h1
    t2_ref[...] = jnp.dot(h1, w2_ref[...], preferred_element_type=jnp.float32)


def _mid(a0, a1, d0, d1, x, root1, b1, w2r):
    blk = 1000
    return pl.pallas_call(
        _mid_body,
        grid=(N // blk,),
        in_specs=[
            pl.BlockSpec((blk, 32), lambda i: (i, 0)),
            pl.BlockSpec((blk, 32), lambda i: (i, 0)),
            pl.BlockSpec((blk, 16), lambda i: (i, 0)),
            pl.BlockSpec((blk, 16), lambda i: (i, 0)),
            pl.BlockSpec((blk, 34), lambda i: (i, 0)),
            pl.BlockSpec((34, 32), lambda i: (0, 0)),
            pl.BlockSpec((1, 32), lambda i: (0, 0)),
            pl.BlockSpec((32, NK * 16), lambda i: (0, 0)),
        ],
        out_specs=[
            pl.BlockSpec((blk, 32), lambda i: (i, 0)),
            pl.BlockSpec((blk, NK * 16), lambda i: (i, 0)),
        ],
        out_shape=[
            jax.ShapeDtypeStruct((N, 32), jnp.float32),
            jax.ShapeDtypeStruct((N, NK * 16), jnp.float32),
        ],
    )(a0, a1, d0, d1, x, root1, b1, w2r)


def _final_body(a0_ref, a1_ref, d0_ref, d1_ref, h1_ref, root_ref, b_ref,
                lw1_ref, lb1_ref, lw2_ref, lb2_ref, o_ref):
    deg = d0_ref[...][:, :1] + d1_ref[...][:, :1]
    inv = 1.0 / jnp.maximum(deg, 1.0)
    agg = (a0_ref[...] + a1_ref[...]) * inv
    h2 = _elu(agg + jnp.dot(h1_ref[...], root_ref[...],
                            preferred_element_type=jnp.float32) + b_ref[...])
    h3 = _elu(jnp.dot(h2, lw1_ref[...],
                      preferred_element_type=jnp.float32) + lb1_ref[...])
    lg = jnp.dot(h3, lw2_ref[...],
                 preferred_element_type=jnp.float32) + lb2_ref[...]
    m = jnp.max(lg, axis=-1, keepdims=True)
    e = jnp.exp(lg - m)
    o_ref[...] = (lg - m) - jnp.log(jnp.sum(e, axis=-1, keepdims=True))


def _final(a0, a1, d0, d1, h1, root2, b2, lw1, lb1, lw2, lb2):
    blk = 1000
    return pl.pallas_call(
        _final_body,
        grid=(N // blk,),
        in_specs=[
            pl.BlockSpec((blk, 16), lambda i: (i, 0)),
            pl.BlockSpec((blk, 16), lambda i: (i, 0)),
            pl.BlockSpec((blk, 16), lambda i: (i, 0)),
            pl.BlockSpec((blk, 16), lambda i: (i, 0)),
            pl.BlockSpec((blk, 32), lambda i: (i, 0)),
            pl.BlockSpec((32, 16), lambda i: (0, 0)),
            pl.BlockSpec((1, 16), lambda i: (0, 0)),
            pl.BlockSpec((16, 32), lambda i: (0, 0)),
            pl.BlockSpec((1, 32), lambda i: (0, 0)),
            pl.BlockSpec((32, 4), lambda i: (0, 0)),
            pl.BlockSpec((1, 4), lambda i: (0, 0)),
        ],
        out_specs=pl.BlockSpec((blk, 4), lambda i: (i, 0)),
        out_shape=jax.ShapeDtypeStruct((N, 4), jnp.float32),
    )(a0, a1, d0, d1, h1, root2, b2, lw1, lb1, lw2, lb2)


def _make_sc_aggregate(out_dim, with_deg):
    mesh = plsc.VectorSubcoreMesh(core_axis_name="c", subcore_axis_name="s")
    H = out_dim // 16

    out_type = [jax.ShapeDtypeStruct((NC, N_PAD, out_dim), jnp.float32)]
    scratch = [
        pltpu.VMEM((8, RCHUNK, LANE), jnp.int32),
        pltpu.VMEM((RCHUNK * LANE * 8,), jnp.float32),
        pltpu.VMEM((RCHUNK, LANE), jnp.int32),
        pltpu.VMEM((8, LANE, out_dim), jnp.float32),
        pltpu.VMEM((LANE, out_dim), jnp.float32),
        pltpu.VMEM_SHARED((N_PAD, out_dim), jnp.float32),
        pltpu.SemaphoreType.DMA,
    ]
    if with_deg:
        out_type.append(jax.ShapeDtypeStruct((NC, N_PAD, 16), jnp.float32))
        scratch.append(pltpu.VMEM((LANE, 16), jnp.float32))
        scratch.append(pltpu.VMEM_SHARED((N_PAD, 16), jnp.float32))

    def body(table_hbm, fidx_hbm, w_hbm, dst_hbm, *rest):
        if with_deg:
            (part_hbm, deg_hbm, fidx_v, w_v, dst_v, rows_v, msg_v, acc_sh,
             sem, ones_v, deg_sh) = rest
        else:
            (part_hbm, fidx_v, w_v, dst_v, rows_v, msg_v, acc_sh,
             sem) = rest
            deg_hbm = ones_v = deg_sh = None
        cid = lax.axis_index("c")
        sid = lax.axis_index("s")

        @pl.loop(0, LANE)
        def _zero(i):
            for h in range(H):
                msg_v[i, pl.ds(16 * h, 16)] = jnp.zeros((16,), jnp.float32)
            if with_deg:
                ones_v[i, :] = jnp.zeros((16,), jnp.float32)

        zbase = sid * NPT
        for z in range(NPT // LANE):
            pltpu.sync_copy(msg_v, acc_sh.at[pl.ds(zbase + z * LANE, LANE)])
            if with_deg:
                pltpu.sync_copy(ones_v,
                                deg_sh.at[pl.ds(zbase + z * LANE, LANE)])
        if with_deg:
            @pl.loop(0, LANE)
            def _ones(i):
                ones_v[i, :] = jnp.ones((16,), jnp.float32)

        plsc.subcore_barrier()

        base_row = cid * (NR_PAD // NC) + sid * ROWS_PER_TILE

        @pl.loop(0, ROWS_PER_TILE // RCHUNK)
        def _chunk(t):
            r0 = base_row + t * RCHUNK
            pltpu.sync_copy(fidx_hbm.at[:, pl.ds(r0, RCHUNK), :], fidx_v)
            pltpu.sync_copy(
                w_hbm.at[pl.ds(r0 * LANE * 8, RCHUNK * LANE * 8)], w_v)
            pltpu.sync_copy(dst_hbm.at[pl.ds(r0, RCHUNK), :], dst_v)
            for j in range(RCHUNK):
                copies = [
                    pltpu.async_copy(table_hbm.at[fidx_v.at[c, j]],
                                     rows_v.at[c], sem)
                    for c in range(8)
                ]
                for cp in copies:
                    cp.wait()

                @pl.loop(0, LANE, unroll=4)
                def _edge(i):
                    ivec = jnp.full((16,), i * 8 + j * (LANE * 8), jnp.int32)
                    ws = [plsc.load_gather(w_v, [ivec + c])
                          for c in range(8)]
                    for h in range(H):
                        sl = pl.ds(16 * h, 16)
                        acc = ws[0] * rows_v[0, i, sl]
                        for c in range(1, 8):
                            acc = acc + ws[c] * rows_v[c, i, sl]
                        msg_v[i, sl] = acc

                pltpu.sync_copy(msg_v, acc_sh.at[dst_v.at[j]], add=True)
                if with_deg:
                    pltpu.sync_copy(ones_v, deg_sh.at[dst_v.at[j]], add=True)

        plsc.subcore_barrier()
        obase = sid * NPT
        pltpu.sync_copy(acc_sh.at[pl.ds(obase, NPT)],
                        part_hbm.at[cid].at[pl.ds(obase, NPT)])
        if with_deg:
            pltpu.sync_copy(deg_sh.at[pl.ds(obase, NPT)],
                            deg_hbm.at[cid].at[pl.ds(obase, NPT)])

    return pl.kernel(body, out_type=out_type, mesh=mesh,
                     scratch_types=scratch,
                     compiler_params=pltpu.CompilerParams(
                         needs_layout_passes=False,
                         use_tc_tiling_on_sc=False))


_sc_layer1 = _make_sc_aggregate(32, True)
_sc_layer2 = _make_sc_aggregate(16, False)


def kernel(x, edge_index, pseudo, W1, root1, b1, W2, root2, b2,
           lw1, lb1, lw2, lb2):
    pad_rows = NR_PAD - NR
    pad2d = ((0, pad_rows), (0, 0))
    src2d = jnp.pad(edge_index[0].reshape(NR, LANE), pad2d)
    p02d = jnp.pad(pseudo[:, 0].reshape(NR, LANE), pad2d)
    p12d = jnp.pad(pseudo[:, 1].reshape(NR, LANE), pad2d)
    p22d = jnp.pad(pseudo[:, 2].reshape(NR, LANE), pad2d)

    fidx3d, w3d = _edge_prep(src2d, p02d, p12d, p22d)
    w_flat = w3d.transpose(1, 2, 0).reshape(E_PAD * 8)
    dst2d = jnp.pad(edge_index[1].reshape(NR, LANE), pad2d,
                    constant_values=N)

    w1r = W1.transpose(1, 0, 2).reshape(34, NK * 32)
    table1 = _table1(x, w1r).reshape(N * NK, 32)

    part1, deg = _sc_layer1(table1, fidx3d, w_flat, dst2d)

    w2r = W2.transpose(1, 0, 2).reshape(32, NK * 16)
    h1, table2 = _mid(part1[0, :N], part1[1, :N], deg[0, :N], deg[1, :N],
                      x, root1, b1.reshape(1, 32), w2r)

    (part2,) = _sc_layer2(table2.reshape(N * NK, 16), fidx3d, w_flat, dst2d)

    return _final(part2[0, :N], part2[1, :N], deg[0, :N], deg[1, :N],
                  h1, root2, b2.reshape(1, 16),
                  lw1, lb1.reshape(1, 32), lw2, lb2.reshape(1, 4))


# speedup vs baseline: 3.6928x; 1.3338x over previous
import functools

import jax
import jax.numpy as jnp
from jax import lax
from jax.experimental import pallas as pl
from jax.experimental.pallas import tpu as pltpu
from jax.experimental.pallas import tpu_sc as plsc

N = 10000
E = 320000
KS = 3
DIM = 3
NK = KS ** DIM
LANE = 128
NR = E // LANE
NR_PAD = 2560
E_PAD = NR_PAD * LANE
N_PAD = 10240
NC = 2
NS = 16
ROWS_PER_TILE = NR_PAD // (NC * NS)
RCHUNK = 4
NPT = N_PAD // NS


def _elu(v):
    return jnp.where(v > 0, v, jnp.exp(jnp.minimum(v, 0.0)) - 1.0)


def _edge_prep_body(src_ref, p0_ref, p1_ref, p2_ref, fidx_ref, w_ref):
    src = src_ref[...]
    ps = (p0_ref[...], p1_ref[...], p2_ref[...])
    i0 = []
    fr = []
    for d in range(DIM):
        u = ps[d] * float(KS - 1)
        i0d = jnp.clip(jnp.floor(u).astype(jnp.int32), 0, KS - 2)
        i0.append(i0d)
        fr.append(u - i0d.astype(jnp.float32))
    base = src * NK + i0[0] + i0[1] * 3 + i0[2] * 9
    for c in range(8):
        offs = [(c >> d) & 1 for d in range(DIM)]
        kadd = offs[0] + offs[1] * 3 + offs[2] * 9
        w = jnp.ones_like(fr[0])
        for d in range(DIM):
            w = w * (fr[d] if offs[d] else (1.0 - fr[d]))
        fidx_ref[c] = base + kadd
        w_ref[c] = w


def _edge_prep(src2d, p02d, p12d, p22d):
    blk = 256
    grid = (NR_PAD // blk,)
    in_spec = pl.BlockSpec((blk, LANE), lambda i: (i, 0))
    out_spec = pl.BlockSpec((8, blk, LANE), lambda i: (0, i, 0))
    return pl.pallas_call(
        _edge_prep_body,
        grid=grid,
        in_specs=[in_spec] * 4,
        out_specs=[out_spec, out_spec],
        out_shape=[
            jax.ShapeDtypeStruct((8, NR_PAD, LANE), jnp.int32),
            jax.ShapeDtypeStruct((8, NR_PAD, LANE), jnp.float32),
        ],
    )(src2d, p02d, p12d, p22d)


def _mm1_body(x_ref, w_ref, o_ref):
    o_ref[...] = jnp.dot(x_ref[...], w_ref[...],
                         preferred_element_type=jnp.float32)


def _table1(x, w1r):
    blk = 1000
    return pl.pallas_call(
        _mm1_body,
        grid=(N // blk,),
        in_specs=[
            pl.BlockSpec((blk, 34), lambda i: (i, 0)),
            pl.BlockSpec((34, NK * 32), lambda i: (0, 0)),
        ],
        out_specs=pl.BlockSpec((blk, NK * 32), lambda i: (i, 0)),
        out_shape=jax.ShapeDtypeStruct((N, NK * 32), jnp.float32),
    )(x, w1r)


def _mid_body(a0_ref, a1_ref, d0_ref, d1_ref, x_ref, root_ref, b_ref,
              w2_ref, h1_ref, t2_ref):
    deg = d0_ref[...][:, :1] + d1_ref[...][:, :1]
    inv = 1.0 / jnp.maximum(deg, 1.0)
    agg = (a0_ref[...] + a1_ref[...]) * inv
    h1 = _elu(agg + jnp.dot(x_ref[...], root_ref[...],
                            preferred_element_type=jnp.float32) + b_ref[...])
    h1_ref[...] = h1
    t2_ref[...] = jnp.dot(h1, w2_ref[...], preferred_element_type=jnp.float32)


def _mid(a0, a1, d0, d1, x, root1, b1, w2r):
    blk = 1000
    return pl.pallas_call(
        _mid_body,
        grid=(N // blk,),
        in_specs=[
            pl.BlockSpec((blk, 32), lambda i: (i, 0)),
            pl.BlockSpec((blk, 32), lambda i: (i, 0)),
            pl.BlockSpec((blk, 16), lambda i: (i, 0)),
            pl.BlockSpec((blk, 16), lambda i: (i, 0)),
            pl.BlockSpec((blk, 34), lambda i: (i, 0)),
            pl.BlockSpec((34, 32), lambda i: (0, 0)),
            pl.BlockSpec((1, 32), lambda i: (0, 0)),
            pl.BlockSpec((32, NK * 16), lambda i: (0, 0)),
        ],
        out_specs=[
            pl.BlockSpec((blk, 32), lambda i: (i, 0)),
            pl.BlockSpec((blk, NK * 16), lambda i: (i, 0)),
        ],
        out_shape=[
            jax.ShapeDtypeStruct((N, 32), jnp.float32),
            jax.ShapeDtypeStruct((N, NK * 16), jnp.float32),
        ],
    )(a0, a1, d0, d1, x, root1, b1, w2r)


def _final_body(a0_ref, a1_ref, d0_ref, d1_ref, h1_ref, root_ref, b_ref,
                lw1_ref, lb1_ref, lw2_ref, lb2_ref, o_ref):
    deg = d0_ref[...][:, :1] + d1_ref[...][:, :1]
    inv = 1.0 / jnp.maximum(deg, 1.0)
    agg = (a0_ref[...] + a1_ref[...]) * inv
    h2 = _elu(agg + jnp.dot(h1_ref[...], root_ref[...],
                            preferred_element_type=jnp.float32) + b_ref[...])
    h3 = _elu(jnp.dot(h2, lw1_ref[...],
                      preferred_element_type=jnp.float32) + lb1_ref[...])
    lg = jnp.dot(h3, lw2_ref[...],
                 preferred_element_type=jnp.float32) + lb2_ref[...]
    m = jnp.max(lg, axis=-1, keepdims=True)
    e = jnp.exp(lg - m)
    o_ref[...] = (lg - m) - jnp.log(jnp.sum(e, axis=-1, keepdims=True))


def _final(a0, a1, d0, d1, h1, root2, b2, lw1, lb1, lw2, lb2):
    blk = 1000
    return pl.pallas_call(
        _final_body,
        grid=(N // blk,),
        in_specs=[
            pl.BlockSpec((blk, 16), lambda i: (i, 0)),
            pl.BlockSpec((blk, 16), lambda i: (i, 0)),
            pl.BlockSpec((blk, 16), lambda i: (i, 0)),
            pl.BlockSpec((blk, 16), lambda i: (i, 0)),
            pl.BlockSpec((blk, 32), lambda i: (i, 0)),
            pl.BlockSpec((32, 16), lambda i: (0, 0)),
            pl.BlockSpec((1, 16), lambda i: (0, 0)),
            pl.BlockSpec((16, 32), lambda i: (0, 0)),
            pl.BlockSpec((1, 32), lambda i: (0, 0)),
            pl.BlockSpec((32, 4), lambda i: (0, 0)),
            pl.BlockSpec((1, 4), lambda i: (0, 0)),
        ],
        out_specs=pl.BlockSpec((blk, 4), lambda i: (i, 0)),
        out_shape=jax.ShapeDtypeStruct((N, 4), jnp.float32),
    )(a0, a1, d0, d1, h1, root2, b2, lw1, lb1, lw2, lb2)


def _make_sc_aggregate(out_dim, with_deg):
    mesh = plsc.VectorSubcoreMesh(core_axis_name="c", subcore_axis_name="s")
    H = out_dim // 16
    NCHUNK = ROWS_PER_TILE // RCHUNK
    WCH = RCHUNK * LANE * 8

    out_type = [jax.ShapeDtypeStruct((NC, N_PAD, out_dim), jnp.float32)]
    scratch = [
        pltpu.VMEM((2, 8, RCHUNK, LANE), jnp.int32),
        pltpu.VMEM((2 * WCH,), jnp.float32),
        pltpu.VMEM((2, RCHUNK, LANE), jnp.int32),
        pltpu.VMEM((2, 8, LANE, out_dim), jnp.float32),
        pltpu.VMEM((LANE, out_dim), jnp.float32),
        pltpu.VMEM_SHARED((N_PAD, out_dim), jnp.float32),
        pltpu.SemaphoreType.DMA,
        pltpu.SemaphoreType.DMA,
    ]
    if with_deg:
        out_type.append(jax.ShapeDtypeStruct((NC, N_PAD, 16), jnp.float32))
        scratch.append(pltpu.VMEM((LANE, 16), jnp.float32))
        scratch.append(pltpu.VMEM_SHARED((N_PAD, 16), jnp.float32))

    def body(table_hbm, fidx_hbm, w_hbm, dst_hbm, *rest):
        if with_deg:
            (part_hbm, deg_hbm, fidx_v, w_v, dst_v, rows_v, msg_v, acc_sh,
             sem_in, sem_g, ones_v, deg_sh) = rest
        else:
            (part_hbm, fidx_v, w_v, dst_v, rows_v, msg_v, acc_sh,
             sem_in, sem_g) = rest
            deg_hbm = ones_v = deg_sh = None
        cid = lax.axis_index("c")
        sid = lax.axis_index("s")

        @pl.loop(0, LANE)
        def _zero(i):
            for h in range(H):
                msg_v[i, pl.ds(16 * h, 16)] = jnp.zeros((16,), jnp.float32)
            if with_deg:
                ones_v[i, :] = jnp.zeros((16,), jnp.float32)

        zbase = sid * NPT
        for z in range(NPT // LANE):
            pltpu.sync_copy(msg_v, acc_sh.at[pl.ds(zbase + z * LANE, LANE)])
            if with_deg:
                pltpu.sync_copy(ones_v,
                                deg_sh.at[pl.ds(zbase + z * LANE, LANE)])
        if with_deg:
            @pl.loop(0, LANE)
            def _ones(i):
                ones_v[i, :] = jnp.ones((16,), jnp.float32)

        plsc.subcore_barrier()

        base_row = cid * (NR_PAD // NC) + sid * ROWS_PER_TILE

        def stage(buf, t):
            r0 = base_row + t * RCHUNK
            pltpu.async_copy(fidx_hbm.at[:, pl.ds(r0, RCHUNK), :],
                             fidx_v.at[buf], sem_in)
            pltpu.async_copy(w_hbm.at[pl.ds(r0 * LANE * 8, WCH)],
                             w_v.at[pl.ds(buf * WCH, WCH)], sem_in)
            pltpu.async_copy(dst_hbm.at[pl.ds(r0, RCHUNK), :],
                             dst_v.at[buf], sem_in)

        def wait_stage(buf):
            pltpu.make_async_copy(fidx_hbm.at[:, pl.ds(0, RCHUNK), :],
                                  fidx_v.at[buf], sem_in).wait()
            pltpu.make_async_copy(w_hbm.at[pl.ds(0, WCH)],
                                  w_v.at[pl.ds(buf * WCH, WCH)],
                                  sem_in).wait()
            pltpu.make_async_copy(dst_hbm.at[pl.ds(0, RCHUNK), :],
                                  dst_v.at[buf], sem_in).wait()

        def fire_gathers(gbuf, ibuf, j):
            for c in range(8):
                pltpu.async_copy(table_hbm.at[fidx_v.at[ibuf, c, j]],
                                 rows_v.at[gbuf, c], sem_g)

        def wait_gathers(gbuf):
            for c in range(8):
                pltpu.make_async_copy(table_hbm.at[fidx_v.at[0, 0, 0]],
                                      rows_v.at[gbuf, c], sem_g).wait()

        stage(0, 0)

        @pl.loop(0, NCHUNK // 2)
        def _chunk(tt):
            for par in range(2):
                t = tt * 2 + par
                wait_stage(par)

                @pl.when(t + 1 < NCHUNK)
                def _():
                    stage(1 - par, t + 1)

                fire_gathers(0, par, 0)
                for j in range(RCHUNK):
                    if j + 1 < RCHUNK:
                        fire_gathers((j + 1) % 2, par, j + 1)
                    wait_gathers(j % 2)
                    gb = j % 2

                    @pl.loop(0, LANE, unroll=4)
                    def _edge(i):
                        ivec = jnp.full(
                            (16,), par * WCH + i * 8 + j * (LANE * 8),
                            jnp.int32)
                        ws = [plsc.load_gather(w_v, [ivec + c])
                              for c in range(8)]
                        for h in range(H):
                            sl = pl.ds(16 * h, 16)
                            acc = ws[0] * rows_v[gb, 0, i, sl]
                            for c in range(1, 8):
                                acc = acc + ws[c] * rows_v[gb, c, i, sl]
                            msg_v[i, sl] = acc

                    pltpu.sync_copy(msg_v, acc_sh.at[dst_v.at[par, j]],
                                    add=True)
                    if with_deg:
                        pltpu.sync_copy(ones_v,
                                        deg_sh.at[dst_v.at[par, j]],
                                        add=True)

        plsc.subcore_barrier()
        obase = sid * NPT
        pltpu.sync_copy(acc_sh.at[pl.ds(obase, NPT)],
                        part_hbm.at[cid].at[pl.ds(obase, NPT)])
        if with_deg:
            pltpu.sync_copy(deg_sh.at[pl.ds(obase, NPT)],
                            deg_hbm.at[cid].at[pl.ds(obase, NPT)])

    return pl.kernel(body, out_type=out_type, mesh=mesh,
                     scratch_types=scratch,
                     compiler_params=pltpu.CompilerParams(
                         needs_layout_passes=False,
                         use_tc_tiling_on_sc=False))


_sc_layer1 = _make_sc_aggregate(32, True)
_sc_layer2 = _make_sc_aggregate(16, False)


def kernel(x, edge_index, pseudo, W1, root1, b1, W2, root2, b2,
           lw1, lb1, lw2, lb2):
    pad_rows = NR_PAD - NR
    pad2d = ((0, pad_rows), (0, 0))
    src2d = jnp.pad(edge_index[0].reshape(NR, LANE), pad2d)
    p02d = jnp.pad(pseudo[:, 0].reshape(NR, LANE), pad2d)
    p12d = jnp.pad(pseudo[:, 1].reshape(NR, LANE), pad2d)
    p22d = jnp.pad(pseudo[:, 2].reshape(NR, LANE), pad2d)

    fidx3d, w3d = _edge_prep(src2d, p02d, p12d, p22d)
    w_flat = w3d.transpose(1, 2, 0).reshape(E_PAD * 8)
    dst2d = jnp.pad(edge_index[1].reshape(NR, LANE), pad2d,
                    constant_values=N)

    w1r = W1.transpose(1, 0, 2).reshape(34, NK * 32)
    table1 = _table1(x, w1r).reshape(N * NK, 32)

    part1, deg = _sc_layer1(table1, fidx3d, w_flat, dst2d)

    w2r = W2.transpose(1, 0, 2).reshape(32, NK * 16)
    h1, table2 = _mid(part1[0, :N], part1[1, :N], deg[0, :N], deg[1, :N],
                      x, root1, b1.reshape(1, 32), w2r)

    (part2,) = _sc_layer2(table2.reshape(N * NK, 16), fidx3d, w_flat, dst2d)

    return _final(part2[0, :N], part2[1, :N], deg[0, :N], deg[1, :N],
                  h1, root2, b2.reshape(1, 16),
                  lw1, lb1.reshape(1, 32), lw2, lb2.reshape(1, 4))


# speedup vs baseline: 4.2702x; 1.1564x over previous
import functools

import jax
import jax.numpy as jnp
from jax import lax
from jax.experimental import pallas as pl
from jax.experimental.pallas import tpu as pltpu
from jax.experimental.pallas import tpu_sc as plsc

N = 10000
E = 320000
KS = 3
DIM = 3
NK = KS ** DIM
LANE = 128
NR = E // LANE
NR_PAD = 2560
E_PAD = NR_PAD * LANE
N_PAD = 10240
NC = 2
NS = 16
ROWS_PER_TILE = NR_PAD // (NC * NS)
RCHUNK = 4
NPT = N_PAD // NS


def _elu(v):
    return jnp.where(v > 0, v, jnp.exp(jnp.minimum(v, 0.0)) - 1.0)


def _edge_prep_body(src_ref, p0_ref, p1_ref, p2_ref, fidx_ref, w_ref):
    src = src_ref[...]
    ps = (p0_ref[...], p1_ref[...], p2_ref[...])
    i0 = []
    fr = []
    for d in range(DIM):
        u = ps[d] * float(KS - 1)
        i0d = jnp.clip(jnp.floor(u).astype(jnp.int32), 0, KS - 2)
        i0.append(i0d)
        fr.append(u - i0d.astype(jnp.float32))
    base = src * NK + i0[0] + i0[1] * 3 + i0[2] * 9
    for c in range(8):
        offs = [(c >> d) & 1 for d in range(DIM)]
        kadd = offs[0] + offs[1] * 3 + offs[2] * 9
        w = jnp.ones_like(fr[0])
        for d in range(DIM):
            w = w * (fr[d] if offs[d] else (1.0 - fr[d]))
        fidx_ref[c] = base + kadd
        w_ref[c] = w


def _edge_prep(src2d, p02d, p12d, p22d):
    blk = 256
    grid = (NR_PAD // blk,)
    in_spec = pl.BlockSpec((blk, LANE), lambda i: (i, 0))
    out_spec = pl.BlockSpec((8, blk, LANE), lambda i: (0, i, 0))
    return pl.pallas_call(
        _edge_prep_body,
        grid=grid,
        in_specs=[in_spec] * 4,
        out_specs=[out_spec, out_spec],
        out_shape=[
            jax.ShapeDtypeStruct((8, NR_PAD, LANE), jnp.int32),
            jax.ShapeDtypeStruct((8, NR_PAD, LANE), jnp.float32),
        ],
    )(src2d, p02d, p12d, p22d)


def _mm1_body(x_ref, w_ref, o_ref):
    o_ref[...] = jnp.dot(x_ref[...], w_ref[...],
                         preferred_element_type=jnp.float32)


def _table1(x, w1r):
    blk = 1000
    return pl.pallas_call(
        _mm1_body,
        grid=(N // blk,),
        in_specs=[
            pl.BlockSpec((blk, 34), lambda i: (i, 0)),
            pl.BlockSpec((34, NK * 32), lambda i: (0, 0)),
        ],
        out_specs=pl.BlockSpec((blk, NK * 32), lambda i: (i, 0)),
        out_shape=jax.ShapeDtypeStruct((N, NK * 32), jnp.float32),
    )(x, w1r)


def _mid_body(a0_ref, a1_ref, x_ref, root_ref, b_ref,
              w2_ref, h1_ref, t2_ref, inv_ref):
    a0 = a0_ref[...]
    a1 = a1_ref[...]
    deg = a0[:, 32:33] + a1[:, 32:33]
    inv = 1.0 / jnp.maximum(deg, 1.0)
    agg = (a0[:, :32] + a1[:, :32]) * inv
    h1 = _elu(agg + jnp.dot(x_ref[...], root_ref[...],
                            preferred_element_type=jnp.float32) + b_ref[...])
    h1_ref[...] = h1
    t2_ref[...] = jnp.dot(h1, w2_ref[...], preferred_element_type=jnp.float32)
    inv_ref[...] = jnp.broadcast_to(inv, (inv.shape[0], 8))


def _mid(a0, a1, x, root1, b1, w2r):
    blk = 1000
    return pl.pallas_call(
        _mid_body,
        grid=(N // blk,),
        in_specs=[
            pl.BlockSpec((blk, 48), lambda i: (i, 0)),
            pl.BlockSpec((blk, 48), lambda i: (i, 0)),
            pl.BlockSpec((blk, 34), lambda i: (i, 0)),
            pl.BlockSpec((34, 32), lambda i: (0, 0)),
            pl.BlockSpec((1, 32), lambda i: (0, 0)),
            pl.BlockSpec((32, NK * 16), lambda i: (0, 0)),
        ],
        out_specs=[
            pl.BlockSpec((blk, 32), lambda i: (i, 0)),
            pl.BlockSpec((blk, NK * 16), lambda i: (i, 0)),
            pl.BlockSpec((blk, 8), lambda i: (i, 0)),
        ],
        out_shape=[
            jax.ShapeDtypeStruct((N, 32), jnp.float32),
            jax.ShapeDtypeStruct((N, NK * 16), jnp.float32),
            jax.ShapeDtypeStruct((N, 8), jnp.float32),
        ],
    )(a0, a1, x, root1, b1, w2r)


def _final_body(a0_ref, a1_ref, inv_ref, h1_ref, root_ref, b_ref,
                lw1_ref, lb1_ref, lw2_ref, lb2_ref, o_ref):
    inv = inv_ref[...][:, :1]
    agg = (a0_ref[...] + a1_ref[...]) * inv
    h2 = _elu(agg + jnp.dot(h1_ref[...], root_ref[...],
                            preferred_element_type=jnp.float32) + b_ref[...])
    h3 = _elu(jnp.dot(h2, lw1_ref[...],
                      preferred_element_type=jnp.float32) + lb1_ref[...])
    lg = jnp.dot(h3, lw2_ref[...],
                 preferred_element_type=jnp.float32) + lb2_ref[...]
    m = jnp.max(lg, axis=-1, keepdims=True)
    e = jnp.exp(lg - m)
    o_ref[...] = (lg - m) - jnp.log(jnp.sum(e, axis=-1, keepdims=True))


def _final(a0, a1, invd, h1, root2, b2, lw1, lb1, lw2, lb2):
    blk = 1000
    return pl.pallas_call(
        _final_body,
        grid=(N // blk,),
        in_specs=[
            pl.BlockSpec((blk, 16), lambda i: (i, 0)),
            pl.BlockSpec((blk, 16), lambda i: (i, 0)),
            pl.BlockSpec((blk, 8), lambda i: (i, 0)),
            pl.BlockSpec((blk, 32), lambda i: (i, 0)),
            pl.BlockSpec((32, 16), lambda i: (0, 0)),
            pl.BlockSpec((1, 16), lambda i: (0, 0)),
            pl.BlockSpec((16, 32), lambda i: (0, 0)),
            pl.BlockSpec((1, 32), lambda i: (0, 0)),
            pl.BlockSpec((32, 4), lambda i: (0, 0)),
            pl.BlockSpec((1, 4), lambda i: (0, 0)),
        ],
        out_specs=pl.BlockSpec((blk, 4), lambda i: (i, 0)),
        out_shape=jax.ShapeDtypeStruct((N, 4), jnp.float32),
    )(a0, a1, invd, h1, root2, b2, lw1, lb1, lw2, lb2)


def _make_sc_aggregate(out_dim, with_deg):
    mesh = plsc.VectorSubcoreMesh(core_axis_name="c", subcore_axis_name="s")
    H = out_dim // 16
    OACC = out_dim + 16 if with_deg else out_dim
    NCHUNK = ROWS_PER_TILE // RCHUNK
    WCH = RCHUNK * LANE

    out_type = [jax.ShapeDtypeStruct((NC, N_PAD, OACC), jnp.float32)]
    scratch = [
        pltpu.VMEM((2, 8, RCHUNK, LANE), jnp.int32),
        pltpu.VMEM((2, 8, WCH), jnp.float32),
        pltpu.VMEM((2, RCHUNK, LANE), jnp.int32),
        pltpu.VMEM((2, 8, LANE, out_dim), jnp.float32),
        pltpu.VMEM((2, LANE, OACC), jnp.float32),
        pltpu.VMEM_SHARED((N_PAD, OACC), jnp.float32),
        pltpu.SemaphoreType.DMA,
        pltpu.SemaphoreType.DMA,
        pltpu.SemaphoreType.DMA,
    ]

    def body(table_hbm, fidx_hbm, w_hbm, dst_hbm, part_hbm,
             fidx_v, w_v, dst_v, rows_v, msg_v, acc_sh,
             sem_in, sem_g, sem_s):
        cid = lax.axis_index("c")
        sid = lax.axis_index("s")

        @pl.loop(0, LANE)
        def _zero(i):
            for b in range(2):
                for h in range(OACC // 16):
                    msg_v[b, i, pl.ds(16 * h, 16)] = jnp.zeros(
                        (16,), jnp.float32)

        zbase = sid * NPT
        for z in range(NPT // LANE):
            pltpu.sync_copy(msg_v.at[0],
                            acc_sh.at[pl.ds(zbase + z * LANE, LANE)])
        if with_deg:
            @pl.loop(0, LANE)
            def _ones(i):
                for b in range(2):
                    msg_v[b, i, pl.ds(out_dim, 16)] = jnp.ones(
                        (16,), jnp.float32)

        plsc.subcore_barrier()

        base_row = cid * (NR_PAD // NC) + sid * ROWS_PER_TILE

        def stage(buf, t):
            r0 = base_row + t * RCHUNK
            pltpu.async_copy(fidx_hbm.at[:, pl.ds(r0, RCHUNK), :],
                             fidx_v.at[buf], sem_in)
            pltpu.async_copy(w_hbm.at[:, pl.ds(r0 * LANE, WCH)],
                             w_v.at[buf], sem_in)
            pltpu.async_copy(dst_hbm.at[pl.ds(r0, RCHUNK), :],
                             dst_v.at[buf], sem_in)

        def wait_stage(buf):
            pltpu.make_async_copy(fidx_hbm.at[:, pl.ds(0, RCHUNK), :],
                                  fidx_v.at[buf], sem_in).wait()
            pltpu.make_async_copy(w_hbm.at[:, pl.ds(0, WCH)],
                                  w_v.at[buf], sem_in).wait()
            pltpu.make_async_copy(dst_hbm.at[pl.ds(0, RCHUNK), :],
                                  dst_v.at[buf], sem_in).wait()

        def fire_gathers(gbuf, ibuf, j):
            for c in range(8):
                pltpu.async_copy(table_hbm.at[fidx_v.at[ibuf, c, j]],
                                 rows_v.at[gbuf, c], sem_g)

        def wait_gathers(gbuf):
            for c in range(8):
                pltpu.make_async_copy(table_hbm.at[fidx_v.at[0, 0, 0]],
                                      rows_v.at[gbuf, c], sem_g).wait()

        def wait_scatter(mb):
            pltpu.make_async_copy(msg_v.at[mb],
                                  acc_sh.at[dst_v.at[0, 0]], sem_s).wait()

        lvecs = [jnp.full((16,), l, jnp.int32) for l in range(16)]
        stage(0, 0)

        @pl.loop(0, NCHUNK // 2)
        def _chunk(tt):
            for par in range(2):
                t = tt * 2 + par
                wait_stage(par)

                @pl.when(t + 1 < NCHUNK)
                def _():
                    stage(1 - par, t + 1)

                fire_gathers(0, par, 0)
                for j in range(RCHUNK):
                    if j + 1 < RCHUNK:
                        fire_gathers((j + 1) % 2, par, j + 1)
                    wait_gathers(j % 2)
                    if j >= 2:
                        wait_scatter(j % 2)
                    gb = j % 2

                    @pl.loop(0, 8)
                    def _grp(g):
                        woff = j * LANE + g * 16
                        wv = [w_v[par, c, pl.ds(woff, 16)] for c in range(8)]

                        @pl.loop(0, 16, unroll=4)
                        def _edge(l):
                            i = g * 16 + l
                            lv = lvecs[0] + l
                            ws = [wv[c].at[lv].get(
                                      mode="promise_in_bounds")
                                  for c in range(8)]
                            for h in range(H):
                                sl = pl.ds(16 * h, 16)
                                acc = ws[0] * rows_v[gb, 0, i, sl]
                                for c in range(1, 8):
                                    acc = acc + ws[c] * rows_v[gb, c, i, sl]
                                msg_v[gb, i, sl] = acc

                    pltpu.async_copy(msg_v.at[gb],
                                     acc_sh.at[dst_v.at[par, j]],
                                     sem_s, add=True)
                wait_scatter(0)
                wait_scatter(1)

        plsc.subcore_barrier()
        obase = sid * NPT
        pltpu.sync_copy(acc_sh.at[pl.ds(obase, NPT)],
                        part_hbm.at[cid].at[pl.ds(obase, NPT)])

    return pl.kernel(body, out_type=out_type, mesh=mesh,
                     scratch_types=scratch,
                     compiler_params=pltpu.CompilerParams(
                         needs_layout_passes=False,
                         use_tc_tiling_on_sc=False))


_sc_layer1 = _make_sc_aggregate(32, True)
_sc_layer2 = _make_sc_aggregate(16, False)


def kernel(x, edge_index, pseudo, W1, root1, b1, W2, root2, b2,
           lw1, lb1, lw2, lb2):
    pad_rows = NR_PAD - NR
    pad2d = ((0, pad_rows), (0, 0))
    src2d = jnp.pad(edge_index[0].reshape(NR, LANE), pad2d)
    p02d = jnp.pad(pseudo[:, 0].reshape(NR, LANE), pad2d)
    p12d = jnp.pad(pseudo[:, 1].reshape(NR, LANE), pad2d)
    p22d = jnp.pad(pseudo[:, 2].reshape(NR, LANE), pad2d)

    fidx3d, w3d = _edge_prep(src2d, p02d, p12d, p22d)
    w2d = w3d.reshape(8, E_PAD)
    dst2d = jnp.pad(edge_index[1].reshape(NR, LANE), pad2d,
                    constant_values=N)

    w1r = W1.transpose(1, 0, 2).reshape(34, NK * 32)
    table1 = _table1(x, w1r).reshape(N * NK, 32)

    (part1,) = _sc_layer1(table1, fidx3d, w2d, dst2d)

    w2r = W2.transpose(1, 0, 2).reshape(32, NK * 16)
    h1, table2, invd = _mid(part1[0, :N], part1[1, :N],
                            x, root1, b1.reshape(1, 32), w2r)

    (part2,) = _sc_layer2(table2.reshape(N * NK, 16), fidx3d, w2d, dst2d)

    return _final(part2[0, :N], part2[1, :N], invd,
                  h1, root2, b2.reshape(1, 16),
                  lw1, lb1.reshape(1, 32), lw2, lb2.reshape(1, 4))
